# baseline (speedup 1.0000x reference)
import jax
import jax.numpy as jnp
from jax.experimental import pallas as pl
from jax.experimental.pallas import tpu as pltpu
from jax.experimental.pallas import tpu_sc as plsc

B = 1024
CTX = 20
D = 64
H = 128
V = 100000
VT = 2000
NV = V // VT
GW = 128


def _sc_gather(idx_flat, table_p):
    mesh = plsc.VectorSubcoreMesh(core_axis_name="core",
                                  subcore_axis_name="subcore")

    @pl.kernel(out_type=jax.ShapeDtypeStruct((B * CTX, 128), jnp.float32),
               mesh=mesh)
    def gather_kernel(tab_hbm, idx_hbm, out_hbm):
        def body(i_vmem, o_vmem):
            pltpu.sync_copy(tab_hbm.at[i_vmem.at[0]], o_vmem)

        pltpu.emit_pipeline(
            body,
            grid=(B * CTX // GW,),
            in_specs=[pl.BlockSpec((1, GW), lambda i: (0, i))],
            out_specs=[pl.BlockSpec((GW, 128), lambda i: (i, 0))],
            core_axis_name=("core", "subcore"),
            dimension_semantics=(pltpu.PARALLEL,),
        )(idx_hbm, out_hbm)

    return gather_kernel(table_p, idx_flat)


def _stats_kernel(g_ref, w1_ref, b1_ref, w2_ref, b2_ref,
                  h_out, lse_out, hs_ref, m_ref, s_ref):
    i = pl.program_id(0)

    @pl.when(i == 0)
    def _():
        emb = jnp.sum(g_ref[...].reshape(CTX, B, 128), axis=0)
        h = jax.lax.dot_general(emb, w1_ref[...],
                                (((1,), (1,)), ((), ())),
                                preferred_element_type=jnp.float32)
        h = jnp.maximum(h + b1_ref[...], 0.0)
        hs_ref[...] = h.astype(jnp.bfloat16)
        m_ref[...] = jnp.full((B, 1), -jnp.inf, jnp.float32)
        s_ref[...] = jnp.zeros((B, 1), jnp.float32)

    w2 = w2_ref[...].astype(jnp.bfloat16)
    logits = jax.lax.dot_general(hs_ref[...], w2,
                                 (((1,), (1,)), ((), ())),
                                 preferred_element_type=jnp.float32)
    logits = logits + b2_ref[...].reshape(1, VT)
    tile_max = jnp.max(logits, axis=1, keepdims=True)
    m_new = jnp.maximum(m_ref[...], tile_max)
    s_ref[...] = (s_ref[...] * jnp.exp(m_ref[...] - m_new)
                  + jnp.sum(jnp.exp(logits - m_new), axis=1, keepdims=True))
    m_ref[...] = m_new

    @pl.when(i == NV - 1)
    def _():
        h_out[...] = hs_ref[...]
        lse_out[...] = jnp.broadcast_to(m_ref[...] + jnp.log(s_ref[...]),
                                        (B, 128))


def _write_kernel(h_ref, lse_ref, w2_ref, b2_ref, o_ref):
    w2 = w2_ref[...].astype(jnp.bfloat16)
    logits = jax.lax.dot_general(h_ref[...], w2,
                                 (((1,), (1,)), ((), ())),
                                 preferred_element_type=jnp.float32)
    out = logits + b2_ref[...].reshape(1, VT) - lse_ref[...][:, 0:1]
    o_ref[...] = out.reshape(B, 1, 1, VT)


def kernel(inputs, table, W1, b1, W2, b2):
    idx_flat = inputs.astype(jnp.int32).T.reshape(1, B * CTX)
    table_p = jnp.pad(table, ((0, 0), (0, 128 - D)))
    gathered = _sc_gather(idx_flat, table_p)

    W1p = jnp.pad(W1, ((0, 0), (0, 128 - D)))
    b1r = b1.reshape(1, H)
    b2r = b2.reshape(NV, 1, VT)

    h, lse = pl.pallas_call(
        _stats_kernel,
        grid=(NV,),
        in_specs=[
            pl.BlockSpec((B * CTX, 128), lambda i: (0, 0)),
            pl.BlockSpec((H, 128), lambda i: (0, 0)),
            pl.BlockSpec((1, H), lambda i: (0, 0)),
            pl.BlockSpec((VT, H), lambda i: (i, 0)),
            pl.BlockSpec((1, 1, VT), lambda i: (i, 0, 0)),
        ],
        out_specs=[
            pl.BlockSpec((B, H), lambda i: (0, 0)),
            pl.BlockSpec((B, 128), lambda i: (0, 0)),
        ],
        out_shape=[
            jax.ShapeDtypeStruct((B, H), jnp.bfloat16),
            jax.ShapeDtypeStruct((B, 128), jnp.float32),
        ],
        scratch_shapes=[
            pltpu.VMEM((B, H), jnp.bfloat16),
            pltpu.VMEM((B, 1), jnp.float32),
            pltpu.VMEM((B, 1), jnp.float32),
        ],
        compiler_params=pltpu.CompilerParams(
            dimension_semantics=("arbitrary",)),
    )(gathered, W1p, b1r, W2, b2r)

    out = pl.pallas_call(
        _write_kernel,
        grid=(NV,),
        in_specs=[
            pl.BlockSpec((B, H), lambda i: (0, 0)),
            pl.BlockSpec((B, 128), lambda i: (0, 0)),
            pl.BlockSpec((VT, H), lambda i: (i, 0)),
            pl.BlockSpec((1, 1, VT), lambda i: (i, 0, 0)),
        ],
        out_specs=pl.BlockSpec((B, 1, 1, VT), lambda i: (0, i, 0, 0)),
        out_shape=jax.ShapeDtypeStruct((B, NV, 1, VT), jnp.float32),
        compiler_params=pltpu.CompilerParams(
            dimension_semantics=("arbitrary",)),
    )(h, lse, W2, b2r)

    return out.reshape(B, V)
